# baseline (speedup 1.0000x reference)
import functools

import jax
import jax.numpy as jnp
from jax import lax
from jax.experimental import pallas as pl
from jax.experimental.pallas import tpu as pltpu
from jax.experimental.pallas import tpu_sc as plsc


_R = 2000


def _max_body(yp_ref, out_ref):
    xt = yp_ref[...].T
    out_ref[0] = jnp.max(xt, axis=0, keepdims=True)


def _row_max(y_pred):
    N, C = y_pred.shape
    grid = N // _R
    out = pl.pallas_call(
        _max_body,
        grid=(grid,),
        in_specs=[pl.BlockSpec((_R, C), lambda i: (i, 0))],
        out_specs=pl.BlockSpec((1, 1, _R), lambda i: (i, 0, 0)),
        out_shape=jax.ShapeDtypeStruct((grid, 1, _R), jnp.float32),
        compiler_params=pltpu.CompilerParams(
            dimension_semantics=("arbitrary",),
        ),
    )(y_pred)
    return out.reshape(N)


_S = 7936
_SROWS = _S // 128
_NSUB = 16
_NLANE = 16


def _sc_body(C, N, ypf, yt_hbm, m_hbm, w_hbm, out_hbm,
             yt_v, m_v, idx_v, vt_v, idxt_v, vtt_v, hist_v, big_v,
             shared_hist, red_v, w_v, dsem):
    wid = lax.axis_index("s")
    nchunks = N // _S
    ntail = N - nchunks * _S
    lane = lax.broadcasted_iota(jnp.int32, (_NLANE,), 0)
    laneC = lane * C
    lane128 = lane * 128
    ones16 = jnp.ones((_NLANE,), jnp.float32)
    zeros16 = jnp.zeros((_NLANE,), jnp.float32)

    def _z(g, _):
        hist_v[pl.ds(g * _NLANE, _NLANE)] = zeros16
        return 0
    lax.fori_loop(0, 4096 // _NLANE, _z, 0)

    nmine = (nchunks - wid + _NSUB - 1) // _NSUB

    def _chunk(k, _):
        cid = wid + k * _NSUB
        base = cid * _S
        pltpu.sync_copy(yt_hbm.at[pl.ds(base, _S)], yt_v)
        pltpu.sync_copy(m_hbm.at[pl.ds(base, _S)], m_v)

        def _mkidx(g, _):
            yt16 = yt_v[pl.ds(g * _NLANE, _NLANE)]
            idx16 = (base + g * _NLANE) * C + laneC + yt16
            idx_v[pl.ds(g * _NLANE, _NLANE)] = idx16
            return 0
        lax.fori_loop(0, _S // _NLANE, _mkidx, 0)

        pltpu.async_copy(ypf.at[idx_v], vt_v, dsem).wait()

        def _acc(g, _):
            yt16 = yt_v[pl.ds(g * _NLANE, _NLANE)]
            m16 = m_v[pl.ds(g * _NLANE, _NLANE)]
            vt16 = vt_v[pl.ds(g * _NLANE, _NLANE)]
            hidx = lane128 + yt16
            valp = jnp.where(vt16 == m16, 1.0, 0.0)
            plsc.addupdate_scatter(hist_v, [hidx], ones16)
            plsc.addupdate_scatter(hist_v, [hidx + 2048], valp)
            return 0
        lax.fori_loop(0, _S // _NLANE, _acc, 0)
        return 0

    lax.fori_loop(0, nmine, _chunk, 0)

    if ntail > 0:
        @pl.when(wid == _NSUB - 1)
        def _tail():
            base = nchunks * _S

            def _zi(g, _):
                idxt_v[pl.ds(g * _NLANE, _NLANE)] = jnp.zeros(
                    (_NLANE,), jnp.int32)
                return 0
            lax.fori_loop(0, 128 // _NLANE, _zi, 0)
            pltpu.sync_copy(yt_hbm.at[pl.ds(base, ntail)],
                            idxt_v.at[pl.ds(0, ntail)])
            pltpu.sync_copy(m_hbm.at[pl.ds(base, ntail)],
                            vtt_v.at[pl.ds(0, ntail)])

            def _mkidx(g, _):
                yt16 = idxt_v[pl.ds(g * _NLANE, _NLANE)]
                idxt_v[pl.ds(g * _NLANE, _NLANE)] = (
                    (base + g * _NLANE) * C + laneC + yt16)
                return 0
            lax.fori_loop(0, ntail // _NLANE, _mkidx, 0)
            pltpu.async_copy(ypf.at[idxt_v], vt_v.at[pl.ds(0, 128)],
                             dsem).wait()

            def _acc(g, _):
                m16 = vtt_v[pl.ds(g * _NLANE, _NLANE)]
                vt16 = vt_v[pl.ds(g * _NLANE, _NLANE)]
                idx16 = idxt_v[pl.ds(g * _NLANE, _NLANE)]
                yt16 = idx16 - (base + g * _NLANE) * C - laneC
                hidx = lane128 + yt16
                valp = jnp.where(vt16 == m16, 1.0, 0.0)
                plsc.addupdate_scatter(hist_v, [hidx], ones16)
                plsc.addupdate_scatter(hist_v, [hidx + 2048], valp)
                return 0
            lax.fori_loop(0, ntail // _NLANE, _acc, 0)

    pltpu.sync_copy(hist_v, shared_hist.at[wid])
    plsc.subcore_barrier()

    @pl.when(wid == 0)
    def _fin():
        pltpu.sync_copy(shared_hist, big_v)
        pltpu.sync_copy(w_hbm, w_v)

        def _cls(c8, carry):
            num16, den16 = carry

            def _slot(i, carry2):
                t16a, p16a = carry2
                t = i // _NSUB
                l = i % _NSUB
                off = l * 128 + c8 * _NLANE
                t16a = t16a + big_v[t, pl.ds(off, _NLANE)]
                p16a = p16a + big_v[t, pl.ds(2048 + off, _NLANE)]
                return (t16a, p16a)
            t16, p16 = lax.fori_loop(0, _NSUB * _NSUB, _slot,
                                     (zeros16, zeros16))
            acc16 = jnp.where(t16 > 0, p16 / jnp.maximum(t16, 1.0), 0.0)
            w16 = w_v[pl.ds(c8 * _NLANE, _NLANE)]
            return (num16 + acc16 * w16, den16 + w16)

        num16, den16 = lax.fori_loop(0, 8, _cls, (zeros16, zeros16))
        num = jnp.broadcast_to(jnp.sum(num16), (_NLANE,))
        den = jnp.broadcast_to(jnp.sum(den16), (_NLANE,))
        red_v[...] = num / den
        pltpu.sync_copy(red_v, out_hbm)
    plsc.subcore_barrier()


def _sc_stage(y_pred, y_true, m, weights_pad):
    N, C = y_pred.shape
    mesh = plsc.VectorSubcoreMesh(
        core_axis_name="c", subcore_axis_name="s", num_cores=1)
    kfn = pl.kernel(
        functools.partial(_sc_body, C, N),
        mesh=mesh,
        out_type=jax.ShapeDtypeStruct((_NLANE,), jnp.float32),
        scratch_types=[
            pltpu.VMEM((_S,), jnp.int32),
            pltpu.VMEM((_S,), jnp.float32),
            pltpu.VMEM((_S,), jnp.int32),
            pltpu.VMEM((_S,), jnp.float32),
            pltpu.VMEM((128,), jnp.int32),
            pltpu.VMEM((128,), jnp.float32),
            pltpu.VMEM((4096,), jnp.float32),
            pltpu.VMEM((_NSUB, 4096), jnp.float32),
            pltpu.VMEM_SHARED((_NSUB, 4096), jnp.float32),
            pltpu.VMEM((_NLANE,), jnp.float32),
            pltpu.VMEM((128,), jnp.float32),
            pltpu.SemaphoreType.DMA,
        ],
        compiler_params=pltpu.CompilerParams(needs_layout_passes=False),
    )
    return kfn(y_pred.reshape(-1), y_true, m, weights_pad)


def kernel(y_pred, y_true, weights):
    N, C = y_pred.shape
    yt = y_true.astype(jnp.int32)
    w_pad = jnp.pad(weights, (0, 128 - C))
    m = _row_max(y_pred)
    out = _sc_stage(y_pred, yt, m, w_pad)
    return out[0]


# speedup vs baseline: 1.9477x; 1.9477x over previous
import functools

import jax
import jax.numpy as jnp
from jax.experimental import pallas as pl
from jax.experimental.pallas import tpu as pltpu

_R = 2000
_NH = 2


def _body(grid, C, yp_ref, yt_ref, w_ref, out_ref, acc_t, acc_p):
    i = pl.program_id(0)

    @pl.when(i == 0)
    def _init():
        acc_t[...] = jnp.zeros_like(acc_t)
        acc_p[...] = jnp.zeros_like(acc_p)

    H = _R // _NH
    x = yp_ref[...]
    yt_all = yt_ref[0]
    for h in range(_NH):
        xt = x[h * H:(h + 1) * H, :].T
        yt = yt_all[:, h * H:(h + 1) * H]
        mt = jnp.max(xt, axis=0, keepdims=True)
        rowid = jax.lax.broadcasted_iota(jnp.int32, (C, H), 0)
        oh_f = jnp.where(rowid == yt, 1.0, 0.0)
        eq_f = jnp.where(xt == mt, 1.0, 0.0)
        acc_t[...] += jax.lax.dot_general(
            oh_f, jnp.ones((H, 1), jnp.float32), (((1,), (0,)), ((), ())),
            preferred_element_type=jnp.float32)
        acc_p[...] += jax.lax.dot_general(
            oh_f, eq_f, (((1,), (1,)), ((), ())),
            preferred_element_type=jnp.float32)

    @pl.when(i == grid - 1)
    def _fin():
        tc = acc_t[...]
        ci = jax.lax.broadcasted_iota(jnp.int32, (C, C), 0)
        cj = jax.lax.broadcasted_iota(jnp.int32, (C, C), 1)
        diag = jnp.where(ci == cj, acc_p[...], 0.0)
        pc = jnp.sum(diag, axis=1, keepdims=True)
        w = w_ref[...]
        acc = jnp.where(tc > 0, pc / jnp.maximum(tc, 1.0), 0.0)
        out_ref[...] = jnp.reshape(jnp.sum(acc * w) / jnp.sum(w), (1, 1))


def kernel(y_pred, y_true, weights):
    N, C = y_pred.shape
    grid = N // _R
    yt3 = y_true.astype(jnp.int32).reshape(grid, 1, _R)
    w2 = weights.reshape(C, 1)
    out = pl.pallas_call(
        functools.partial(_body, grid, C),
        grid=(grid,),
        in_specs=[
            pl.BlockSpec((_R, C), lambda i: (i, 0)),
            pl.BlockSpec((1, 1, _R), lambda i: (i, 0, 0)),
            pl.BlockSpec((C, 1), lambda i: (0, 0)),
        ],
        out_specs=pl.BlockSpec((1, 1), lambda i: (0, 0)),
        out_shape=jax.ShapeDtypeStruct((1, 1), jnp.float32),
        scratch_shapes=[
            pltpu.VMEM((C, 1), jnp.float32),
            pltpu.VMEM((C, C), jnp.float32),
        ],
        compiler_params=pltpu.CompilerParams(
            dimension_semantics=("arbitrary",),
        ),
    )(y_pred, yt3, w2)
    return out[0, 0]


# speedup vs baseline: 2.3327x; 1.1977x over previous
import functools

import jax
import jax.numpy as jnp
from jax.experimental import pallas as pl
from jax.experimental.pallas import tpu as pltpu

_R = 4000
_NH = 2


def _body(grid, C, yp_ref, yt_ref, w_ref, out_ref, acc_t, acc_p):
    i = pl.program_id(0)

    @pl.when(i == 0)
    def _init():
        acc_t[...] = jnp.zeros_like(acc_t)
        acc_p[...] = jnp.zeros_like(acc_p)

    H = _R // _NH
    x = yp_ref[...]
    yt_all = yt_ref[0]
    for h in range(_NH):
        xt = x[h * H:(h + 1) * H, :].T
        yt = yt_all[:, h * H:(h + 1) * H]
        mt = jnp.max(xt, axis=0, keepdims=True)
        rowid = jax.lax.broadcasted_iota(jnp.int32, (C, H), 0)
        oh_f = jnp.where(rowid == yt, 1.0, 0.0)
        eq_f = jnp.where(xt == mt, 1.0, 0.0)
        acc_t[...] += jax.lax.dot_general(
            oh_f, jnp.ones((H, 1), jnp.float32), (((1,), (0,)), ((), ())),
            preferred_element_type=jnp.float32)
        acc_p[...] += jax.lax.dot_general(
            oh_f, eq_f, (((1,), (1,)), ((), ())),
            preferred_element_type=jnp.float32)

    @pl.when(i == grid - 1)
    def _fin():
        tc = acc_t[...]
        ci = jax.lax.broadcasted_iota(jnp.int32, (C, C), 0)
        cj = jax.lax.broadcasted_iota(jnp.int32, (C, C), 1)
        diag = jnp.where(ci == cj, acc_p[...], 0.0)
        pc = jnp.sum(diag, axis=1, keepdims=True)
        w = w_ref[...]
        acc = jnp.where(tc > 0, pc / jnp.maximum(tc, 1.0), 0.0)
        out_ref[...] = jnp.reshape(jnp.sum(acc * w) / jnp.sum(w), (1, 1))


def kernel(y_pred, y_true, weights):
    N, C = y_pred.shape
    grid = N // _R
    yt3 = y_true.astype(jnp.int32).reshape(grid, 1, _R)
    w2 = weights.reshape(C, 1)
    out = pl.pallas_call(
        functools.partial(_body, grid, C),
        grid=(grid,),
        in_specs=[
            pl.BlockSpec((_R, C), lambda i: (i, 0)),
            pl.BlockSpec((1, 1, _R), lambda i: (i, 0, 0)),
            pl.BlockSpec((C, 1), lambda i: (0, 0)),
        ],
        out_specs=pl.BlockSpec((1, 1), lambda i: (0, 0)),
        out_shape=jax.ShapeDtypeStruct((1, 1), jnp.float32),
        scratch_shapes=[
            pltpu.VMEM((C, 1), jnp.float32),
            pltpu.VMEM((C, C), jnp.float32),
        ],
        compiler_params=pltpu.CompilerParams(
            dimension_semantics=("arbitrary",),
        ),
    )(y_pred, yt3, w2)
    return out[0, 0]


# speedup vs baseline: 2.6114x; 1.1195x over previous
import functools

import jax
import jax.numpy as jnp
from jax.experimental import pallas as pl
from jax.experimental.pallas import tpu as pltpu

_R = 8000
_NH = 2


def _body(grid, C, yp_ref, yt_ref, w_ref, out_ref, acc_t, acc_p):
    i = pl.program_id(0)

    @pl.when(i == 0)
    def _init():
        acc_t[...] = jnp.zeros_like(acc_t)
        acc_p[...] = jnp.zeros_like(acc_p)

    H = _R // _NH
    x = yp_ref[...]
    yt_all = yt_ref[0]
    for h in range(_NH):
        xt = x[h * H:(h + 1) * H, :].T
        yt = yt_all[:, h * H:(h + 1) * H]
        mt = jnp.max(xt, axis=0, keepdims=True)
        rowid = jax.lax.broadcasted_iota(jnp.int32, (C, H), 0)
        oh_f = jnp.where(rowid == yt, 1.0, 0.0)
        eq_f = jnp.where(xt == mt, 1.0, 0.0)
        acc_t[...] += jax.lax.dot_general(
            oh_f, jnp.ones((H, 1), jnp.float32), (((1,), (0,)), ((), ())),
            preferred_element_type=jnp.float32)
        acc_p[...] += jax.lax.dot_general(
            oh_f, eq_f, (((1,), (1,)), ((), ())),
            preferred_element_type=jnp.float32)

    @pl.when(i == grid - 1)
    def _fin():
        tc = acc_t[...]
        ci = jax.lax.broadcasted_iota(jnp.int32, (C, C), 0)
        cj = jax.lax.broadcasted_iota(jnp.int32, (C, C), 1)
        diag = jnp.where(ci == cj, acc_p[...], 0.0)
        pc = jnp.sum(diag, axis=1, keepdims=True)
        w = w_ref[...]
        acc = jnp.where(tc > 0, pc / jnp.maximum(tc, 1.0), 0.0)
        out_ref[...] = jnp.reshape(jnp.sum(acc * w) / jnp.sum(w), (1, 1))


def kernel(y_pred, y_true, weights):
    N, C = y_pred.shape
    grid = N // _R
    yt3 = y_true.astype(jnp.int32).reshape(grid, 1, _R)
    w2 = weights.reshape(C, 1)
    out = pl.pallas_call(
        functools.partial(_body, grid, C),
        grid=(grid,),
        in_specs=[
            pl.BlockSpec((_R, C), lambda i: (i, 0)),
            pl.BlockSpec((1, 1, _R), lambda i: (i, 0, 0)),
            pl.BlockSpec((C, 1), lambda i: (0, 0)),
        ],
        out_specs=pl.BlockSpec((1, 1), lambda i: (0, 0)),
        out_shape=jax.ShapeDtypeStruct((1, 1), jnp.float32),
        scratch_shapes=[
            pltpu.VMEM((C, 1), jnp.float32),
            pltpu.VMEM((C, C), jnp.float32),
        ],
        compiler_params=pltpu.CompilerParams(
            dimension_semantics=("arbitrary",),
        ),
    )(y_pred, yt3, w2)
    return out[0, 0]


# speedup vs baseline: 2.8067x; 1.0748x over previous
import functools

import jax
import jax.numpy as jnp
from jax.experimental import pallas as pl
from jax.experimental.pallas import tpu as pltpu

_R = 20000
_NH = 2


def _body(grid, C, yp_ref, yt_ref, w_ref, out_ref, acc_t, acc_p):
    i = pl.program_id(0)

    @pl.when(i == 0)
    def _init():
        acc_t[...] = jnp.zeros_like(acc_t)
        acc_p[...] = jnp.zeros_like(acc_p)

    H = _R // _NH
    x = yp_ref[...]
    yt_all = yt_ref[0]
    for h in range(_NH):
        xt = x[h * H:(h + 1) * H, :].T
        yt = yt_all[:, h * H:(h + 1) * H]
        mt = jnp.max(xt, axis=0, keepdims=True)
        rowid = jax.lax.broadcasted_iota(jnp.int32, (C, H), 0)
        oh_f = jnp.where(rowid == yt, 1.0, 0.0)
        eq_f = jnp.where(xt == mt, 1.0, 0.0)
        acc_t[...] += jax.lax.dot_general(
            oh_f, jnp.ones((H, 1), jnp.float32), (((1,), (0,)), ((), ())),
            preferred_element_type=jnp.float32)
        acc_p[...] += jax.lax.dot_general(
            oh_f, eq_f, (((1,), (1,)), ((), ())),
            preferred_element_type=jnp.float32)

    @pl.when(i == grid - 1)
    def _fin():
        tc = acc_t[...]
        ci = jax.lax.broadcasted_iota(jnp.int32, (C, C), 0)
        cj = jax.lax.broadcasted_iota(jnp.int32, (C, C), 1)
        diag = jnp.where(ci == cj, acc_p[...], 0.0)
        pc = jnp.sum(diag, axis=1, keepdims=True)
        w = w_ref[...]
        acc = jnp.where(tc > 0, pc / jnp.maximum(tc, 1.0), 0.0)
        out_ref[...] = jnp.reshape(jnp.sum(acc * w) / jnp.sum(w), (1, 1))


def kernel(y_pred, y_true, weights):
    N, C = y_pred.shape
    grid = N // _R
    yt3 = y_true.astype(jnp.int32).reshape(grid, 1, _R)
    w2 = weights.reshape(C, 1)
    out = pl.pallas_call(
        functools.partial(_body, grid, C),
        grid=(grid,),
        in_specs=[
            pl.BlockSpec((_R, C), lambda i: (i, 0)),
            pl.BlockSpec((1, 1, _R), lambda i: (i, 0, 0)),
            pl.BlockSpec((C, 1), lambda i: (0, 0)),
        ],
        out_specs=pl.BlockSpec((1, 1), lambda i: (0, 0)),
        out_shape=jax.ShapeDtypeStruct((1, 1), jnp.float32),
        scratch_shapes=[
            pltpu.VMEM((C, 1), jnp.float32),
            pltpu.VMEM((C, C), jnp.float32),
        ],
        compiler_params=pltpu.CompilerParams(
            dimension_semantics=("arbitrary",),
        ),
    )(y_pred, yt3, w2)
    return out[0, 0]


# speedup vs baseline: 2.8413x; 1.0123x over previous
import functools

import jax
import jax.numpy as jnp
from jax.experimental import pallas as pl
from jax.experimental.pallas import tpu as pltpu

_R = 40000
_NH = 2


def _body(grid, C, yp_ref, yt_ref, w_ref, out_ref, acc_t, acc_p):
    i = pl.program_id(0)

    @pl.when(i == 0)
    def _init():
        acc_t[...] = jnp.zeros_like(acc_t)
        acc_p[...] = jnp.zeros_like(acc_p)

    H = _R // _NH
    x = yp_ref[...]
    yt_all = yt_ref[0]
    for h in range(_NH):
        xt = x[h * H:(h + 1) * H, :].T
        yt = yt_all[:, h * H:(h + 1) * H]
        mt = jnp.max(xt, axis=0, keepdims=True)
        rowid = jax.lax.broadcasted_iota(jnp.int32, (C, H), 0)
        oh_f = jnp.where(rowid == yt, 1.0, 0.0)
        eq_f = jnp.where(xt == mt, 1.0, 0.0)
        acc_t[...] += jax.lax.dot_general(
            oh_f, jnp.ones((H, 1), jnp.float32), (((1,), (0,)), ((), ())),
            preferred_element_type=jnp.float32)
        acc_p[...] += jax.lax.dot_general(
            oh_f, eq_f, (((1,), (1,)), ((), ())),
            preferred_element_type=jnp.float32)

    @pl.when(i == grid - 1)
    def _fin():
        tc = acc_t[...]
        ci = jax.lax.broadcasted_iota(jnp.int32, (C, C), 0)
        cj = jax.lax.broadcasted_iota(jnp.int32, (C, C), 1)
        diag = jnp.where(ci == cj, acc_p[...], 0.0)
        pc = jnp.sum(diag, axis=1, keepdims=True)
        w = w_ref[...]
        acc = jnp.where(tc > 0, pc / jnp.maximum(tc, 1.0), 0.0)
        out_ref[...] = jnp.reshape(jnp.sum(acc * w) / jnp.sum(w), (1, 1))


def kernel(y_pred, y_true, weights):
    N, C = y_pred.shape
    grid = N // _R
    yt3 = y_true.astype(jnp.int32).reshape(grid, 1, _R)
    w2 = weights.reshape(C, 1)
    out = pl.pallas_call(
        functools.partial(_body, grid, C),
        grid=(grid,),
        in_specs=[
            pl.BlockSpec((_R, C), lambda i: (i, 0)),
            pl.BlockSpec((1, 1, _R), lambda i: (i, 0, 0)),
            pl.BlockSpec((C, 1), lambda i: (0, 0)),
        ],
        out_specs=pl.BlockSpec((1, 1), lambda i: (0, 0)),
        out_shape=jax.ShapeDtypeStruct((1, 1), jnp.float32),
        scratch_shapes=[
            pltpu.VMEM((C, 1), jnp.float32),
            pltpu.VMEM((C, C), jnp.float32),
        ],
        compiler_params=pltpu.CompilerParams(
            dimension_semantics=("arbitrary",),
        ),
    )(y_pred, yt3, w2)
    return out[0, 0]
